# baseline (speedup 1.0000x reference)
import functools

import jax
import jax.numpy as jnp
from jax import lax
from jax.experimental import pallas as pl
from jax.experimental.pallas import tpu as pltpu
from jax.experimental.pallas import tpu_sc as plsc

VOCAB = 100000
EMBED = 128
BATCH = 16384

_info = plsc.get_sparse_core_info()
_NC, _NS = _info.num_cores, _info.num_subcores
NW = _NC * _NS
CHUNK = 64
PAIRS_PER_W = BATCH // NW
NCH = PAIRS_PER_W // CHUNK
NSLC = EMBED // 16
NBUF = 2

_mesh = plsc.VectorSubcoreMesh(core_axis_name="c", subcore_axis_name="s")

_GDN = lax.GatherDimensionNumbers(
    offset_dims=(), collapsed_slice_dims=(0,), start_index_map=(0,))


def _lane_perm(v, idx):
    return lax.gather(v, idx[:, None], _GDN, slice_sizes=(1,),
                      mode=lax.GatherScatterMode.PROMISE_IN_BOUNDS)


_scratch = (
    [pltpu.VMEM((NCH, CHUNK), jnp.int32) for _ in range(2)]
    + [pltpu.VMEM((CHUNK, EMBED), jnp.float32) for _ in range(NBUF)]
    + [pltpu.VMEM((CHUNK, EMBED), jnp.float32) for _ in range(NBUF)]
    + [pltpu.VMEM((CHUNK,), jnp.float32) for _ in range(NBUF)]
    + [pltpu.VMEM((CHUNK,), jnp.float32) for _ in range(NBUF)]
    + [pltpu.VMEM((PAIRS_PER_W,), jnp.float32)]
    + [pltpu.VMEM((16, 16), jnp.float32)]
    + [pltpu.SemaphoreType.DMA for _ in range(NBUF)]
)


@functools.partial(
    pl.kernel,
    out_type=jax.ShapeDtypeStruct((BATCH,), jnp.float32),
    mesh=_mesh,
    scratch_types=_scratch,
)
def _glove_sc(word_u, word_v, in_embed, in_bias, out_embed, out_bias, out,
              *scratch):
    idx_u, idx_v = scratch[0:2]
    rest = scratch[2:]
    rows_u = rest[0:NBUF]
    rows_v = rest[NBUF:2 * NBUF]
    bias_u = rest[2 * NBUF:3 * NBUF]
    bias_v = rest[3 * NBUF:4 * NBUF]
    out_buf = rest[4 * NBUF]
    psbuf = rest[4 * NBUF + 1]
    sem = rest[4 * NBUF + 2:4 * NBUF + 2 + NBUF]

    wid = lax.axis_index("s") * _NC + lax.axis_index("c")
    lane = lax.iota(jnp.int32, 16)
    perm_idx = [lane ^ b for b in (1, 2, 4, 8)]
    lo_mask = [(lane & b) == 0 for b in (1, 2, 4, 8)]

    pltpu.sync_copy(word_u.at[wid], idx_u)
    pltpu.sync_copy(word_v.at[wid], idx_v)

    def issue(c):
        p = c % NBUF
        return [
            pltpu.async_copy(in_embed.at[idx_u.at[c]], rows_u[p], sem[p]),
            pltpu.async_copy(out_embed.at[idx_v.at[c]], rows_v[p], sem[p]),
            pltpu.async_copy(in_bias.at[idx_u.at[c]], bias_u[p], sem[p]),
            pltpu.async_copy(out_bias.at[idx_v.at[c]], bias_v[p], sem[p]),
        ]

    pending = {c: issue(c) for c in range(min(NBUF - 1, NCH))}
    for c in range(NCH):
        p = c % NBUF
        nxt = c + NBUF - 1
        if nxt < NCH:
            pending[nxt] = issue(nxt)
        for d in pending.pop(c):
            d.wait()
        ru, rv, bu, bv = rows_u[p], rows_v[p], bias_u[p], bias_v[p]

        def group_body(g, carry, rows_u=ru, rows_v=rv, bias_u=bu, bias_v=bv,
                       c=c):
            base = g * 16
            for i in range(16):
                r = base + i
                ts = [rows_u[r, pl.ds(k * 16, 16)] * rows_v[r, pl.ds(k * 16, 16)]
                      for k in range(NSLC)]
                while len(ts) > 1:
                    ts = [ts[j] + ts[j + 1] for j in range(0, len(ts), 2)]
                psbuf[i, pl.ds(0, 16)] = ts[0]
            lvl = [None] * 5
            for i in range(16):
                cur = psbuf[i, pl.ds(0, 16)]
                s = 0
                while lvl[s] is not None:
                    a = lvl[s]
                    lvl[s] = None
                    cur = jnp.where(lo_mask[s],
                                    a + _lane_perm(a, perm_idx[s]),
                                    cur + _lane_perm(cur, perm_idx[s]))
                    s += 1
                lvl[s] = cur
            tot = lvl[4] + bias_u[pl.ds(base, 16)] + bias_v[pl.ds(base, 16)]
            out_buf[pl.ds(c * CHUNK + base, 16)] = tot
            return carry

        lax.fori_loop(0, CHUNK // 16, group_body, 0)

    pltpu.sync_copy(out_buf, out.at[pl.ds(wid * PAIRS_PER_W, PAIRS_PER_W)])


def kernel(word_u, word_v, in_embed, in_bias, out_embed, out_bias):
    wu = word_u.reshape(NW, NCH, CHUNK)
    wv = word_v.reshape(NW, NCH, CHUNK)
    return _glove_sc(wu, wv, in_embed, in_bias.reshape(VOCAB),
                     out_embed, out_bias.reshape(VOCAB))


# speedup vs baseline: 1.1271x; 1.1271x over previous
import functools

import jax
import jax.numpy as jnp
from jax import lax
from jax.experimental import pallas as pl
from jax.experimental.pallas import tpu as pltpu
from jax.experimental.pallas import tpu_sc as plsc

VOCAB = 100000
EMBED = 128
BATCH = 16384

_info = plsc.get_sparse_core_info()
_NC, _NS = _info.num_cores, _info.num_subcores
NW = _NC * _NS
CHUNK = 128
PAIRS_PER_W = BATCH // NW
NCH = PAIRS_PER_W // CHUNK
NSLC = EMBED // 16
NBUF = 2

_mesh = plsc.VectorSubcoreMesh(core_axis_name="c", subcore_axis_name="s")

_GDN = lax.GatherDimensionNumbers(
    offset_dims=(), collapsed_slice_dims=(0,), start_index_map=(0,))


def _lane_perm(v, idx):
    return lax.gather(v, idx[:, None], _GDN, slice_sizes=(1,),
                      mode=lax.GatherScatterMode.PROMISE_IN_BOUNDS)


_scratch = (
    [pltpu.VMEM((NCH, CHUNK), jnp.int32) for _ in range(2)]
    + [pltpu.VMEM((CHUNK, EMBED), jnp.float32) for _ in range(NBUF)]
    + [pltpu.VMEM((CHUNK, EMBED), jnp.float32) for _ in range(NBUF)]
    + [pltpu.VMEM((CHUNK,), jnp.float32) for _ in range(NBUF)]
    + [pltpu.VMEM((CHUNK,), jnp.float32) for _ in range(NBUF)]
    + [pltpu.VMEM((PAIRS_PER_W,), jnp.float32)]
    + [pltpu.VMEM((16, 16), jnp.float32)]
    + [pltpu.SemaphoreType.DMA for _ in range(NBUF)]
)


@functools.partial(
    pl.kernel,
    out_type=jax.ShapeDtypeStruct((BATCH,), jnp.float32),
    mesh=_mesh,
    scratch_types=_scratch,
)
def _glove_sc(word_u, word_v, in_embed, in_bias, out_embed, out_bias, out,
              *scratch):
    idx_u, idx_v = scratch[0:2]
    rest = scratch[2:]
    rows_u = rest[0:NBUF]
    rows_v = rest[NBUF:2 * NBUF]
    bias_u = rest[2 * NBUF:3 * NBUF]
    bias_v = rest[3 * NBUF:4 * NBUF]
    out_buf = rest[4 * NBUF]
    psbuf = rest[4 * NBUF + 1]
    sem = rest[4 * NBUF + 2:4 * NBUF + 2 + NBUF]

    wid = lax.axis_index("s") * _NC + lax.axis_index("c")
    lane = lax.iota(jnp.int32, 16)
    perm_idx = [lane ^ b for b in (1, 2, 4, 8)]
    lo_mask = [(lane & b) == 0 for b in (1, 2, 4, 8)]

    pltpu.sync_copy(word_u.at[wid], idx_u)
    pltpu.sync_copy(word_v.at[wid], idx_v)

    def issue(c):
        p = c % NBUF
        return [
            pltpu.async_copy(in_embed.at[idx_u.at[c]], rows_u[p], sem[p]),
            pltpu.async_copy(out_embed.at[idx_v.at[c]], rows_v[p], sem[p]),
            pltpu.async_copy(in_bias.at[idx_u.at[c]], bias_u[p], sem[p]),
            pltpu.async_copy(out_bias.at[idx_v.at[c]], bias_v[p], sem[p]),
        ]

    pending = {c: issue(c) for c in range(min(NBUF - 1, NCH))}
    for c in range(NCH):
        p = c % NBUF
        nxt = c + NBUF - 1
        if nxt < NCH:
            pending[nxt] = issue(nxt)
        for d in pending.pop(c):
            d.wait()
        ru, rv, bu, bv = rows_u[p], rows_v[p], bias_u[p], bias_v[p]

        def group_body(g, carry, rows_u=ru, rows_v=rv, bias_u=bu, bias_v=bv,
                       c=c):
            base = g * 16
            for i in range(16):
                r = base + i
                ts = [rows_u[r, pl.ds(k * 16, 16)] * rows_v[r, pl.ds(k * 16, 16)]
                      for k in range(NSLC)]
                while len(ts) > 1:
                    ts = [ts[j] + ts[j + 1] for j in range(0, len(ts), 2)]
                psbuf[i, pl.ds(0, 16)] = ts[0]
            lvl = [None] * 5
            for i in range(16):
                cur = psbuf[i, pl.ds(0, 16)]
                s = 0
                while lvl[s] is not None:
                    a = lvl[s]
                    lvl[s] = None
                    cur = jnp.where(lo_mask[s],
                                    a + _lane_perm(a, perm_idx[s]),
                                    cur + _lane_perm(cur, perm_idx[s]))
                    s += 1
                lvl[s] = cur
            tot = lvl[4] + bias_u[pl.ds(base, 16)] + bias_v[pl.ds(base, 16)]
            out_buf[pl.ds(c * CHUNK + base, 16)] = tot
            return carry

        lax.fori_loop(0, CHUNK // 16, group_body, 0)

    pltpu.sync_copy(out_buf, out.at[pl.ds(wid * PAIRS_PER_W, PAIRS_PER_W)])


def kernel(word_u, word_v, in_embed, in_bias, out_embed, out_bias):
    wu = word_u.reshape(NW, NCH, CHUNK)
    wv = word_v.reshape(NW, NCH, CHUNK)
    return _glove_sc(wu, wv, in_embed, in_bias.reshape(VOCAB),
                     out_embed, out_bias.reshape(VOCAB))
